# Initial kernel scaffold; baseline (speedup 1.0000x reference)
import jax
import jax.numpy as jnp
from jax.experimental import pallas as pl


def kernel(x, table):
    raise NotImplementedError("write your pallas kernel here")


# speedup vs baseline: 1.6836x; 1.6836x over previous
import functools

import jax
import jax.numpy as jnp
from jax import lax
from jax.experimental import pallas as pl
from jax.experimental.pallas import tpu as pltpu
from jax.experimental.pallas import tpu_sc as plsc

NUM_WORKERS = 32
CHUNK = 128
B_TOTAL = 16384 * 50
B_PER_W = B_TOTAL // NUM_WORKERS
N_CHUNKS = B_PER_W // CHUNK
DIM = 64


def _embedding_lookup(x_grouped, table):
    mesh = plsc.VectorSubcoreMesh(core_axis_name="c", subcore_axis_name="s")

    @functools.partial(
        pl.kernel,
        out_type=jax.ShapeDtypeStruct((B_TOTAL, DIM), jnp.float32),
        mesh=mesh,
        scratch_types=[
            pltpu.VMEM((N_CHUNKS, CHUNK), jnp.int32),
            pltpu.VMEM((CHUNK, DIM), jnp.float32),
            pltpu.SemaphoreType.DMA,
        ],
        compiler_params=pltpu.CompilerParams(use_tc_tiling_on_sc=False),
    )
    def body(idx_hbm, table_hbm, out_hbm, idx_v, rows_v, sem):
        wid = lax.axis_index("s") * 2 + lax.axis_index("c")
        base = wid * B_PER_W
        pltpu.sync_copy(idx_hbm.at[wid], idx_v)

        def step(j, carry):
            pltpu.async_copy(table_hbm.at[idx_v.at[j]], rows_v, sem).wait()
            pltpu.sync_copy(rows_v, out_hbm.at[pl.ds(base + j * CHUNK, CHUNK)])
            return carry

        lax.fori_loop(0, N_CHUNKS, step, 0)

    return body(x_grouped, table)


def kernel(x, table):
    x_grouped = x.reshape(NUM_WORKERS, N_CHUNKS, CHUNK).astype(jnp.int32)
    out_flat = _embedding_lookup(x_grouped, table)
    return out_flat.reshape(x.shape[0], x.shape[1], DIM)


# speedup vs baseline: 1.8764x; 1.1145x over previous
import functools

import jax
import jax.numpy as jnp
from jax import lax
from jax.experimental import pallas as pl
from jax.experimental.pallas import tpu as pltpu
from jax.experimental.pallas import tpu_sc as plsc

NUM_WORKERS = 32
CHUNK = 128
B_TOTAL = 16384 * 50
B_PER_W = B_TOTAL // NUM_WORKERS
N_CHUNKS = B_PER_W // CHUNK
DIM = 64


NBUF = 4


def _embedding_lookup(x_grouped, table):
    mesh = plsc.VectorSubcoreMesh(core_axis_name="c", subcore_axis_name="s")

    @functools.partial(
        pl.kernel,
        out_type=jax.ShapeDtypeStruct((B_TOTAL, DIM), jnp.float32),
        mesh=mesh,
        scratch_types=[
            pltpu.VMEM((N_CHUNKS, CHUNK), jnp.int32),
            pltpu.VMEM((NBUF, CHUNK, DIM), jnp.float32),
            pltpu.SemaphoreType.DMA((NBUF,)),
            pltpu.SemaphoreType.DMA((NBUF,)),
        ],
        compiler_params=pltpu.CompilerParams(use_tc_tiling_on_sc=False),
    )
    def body(idx_hbm, table_hbm, out_hbm, idx_v, rows_v, g_sem, o_sem):
        wid = lax.axis_index("s") * 2 + lax.axis_index("c")
        base = wid * B_PER_W
        pltpu.sync_copy(idx_hbm.at[wid], idx_v)

        def start_gather(j):
            b = lax.rem(j, NBUF)
            pltpu.async_copy(table_hbm.at[idx_v.at[j]], rows_v.at[b],
                             g_sem.at[b])

        def wait_gather(j):
            b = lax.rem(j, NBUF)
            pltpu.make_async_copy(table_hbm.at[idx_v.at[j]], rows_v.at[b],
                                  g_sem.at[b]).wait()

        def start_out(j):
            b = lax.rem(j, NBUF)
            pltpu.async_copy(rows_v.at[b],
                             out_hbm.at[pl.ds(base + j * CHUNK, CHUNK)],
                             o_sem.at[b])

        def wait_out(j):
            b = lax.rem(j, NBUF)
            pltpu.make_async_copy(rows_v.at[b],
                                  out_hbm.at[pl.ds(base + j * CHUNK, CHUNK)],
                                  o_sem.at[b]).wait()

        for j in range(NBUF - 1):
            start_gather(j)

        def step(i, carry):
            wait_gather(i)
            start_out(i)

            @pl.when(i > 0)
            def _():
                wait_out(i - 1)

            @pl.when(i + NBUF - 1 < N_CHUNKS)
            def _():
                start_gather(i + NBUF - 1)

            return carry

        lax.fori_loop(0, N_CHUNKS, step, 0)
        wait_out(N_CHUNKS - 1)

    return body(x_grouped, table)


def kernel(x, table):
    x_grouped = x.reshape(NUM_WORKERS, N_CHUNKS, CHUNK).astype(jnp.int32)
    out_flat = _embedding_lookup(x_grouped, table)
    return out_flat.reshape(x.shape[0], x.shape[1], DIM)
